# baseline (speedup 1.0000x reference)
import jax
import jax.numpy as jnp
from jax import lax
from jax.experimental import pallas as pl
from jax.experimental.pallas import tpu as pltpu
from jax.experimental.pallas import tpu_sc as plsc

VOCAB = 1000000
DIM = 64
NUM_NS = 5
BATCH = 16384

NT = NUM_NS + 1
NC, NS = 2, 16
NW = NC * NS
B_PER_W = BATCH // NW
CB = 128
NCHUNK = B_PER_W // CB
GRP = CB // 16


def _cbow_body(ctx_idx, tgt_idx, ctx_table, tgt_table, out,
               idx_c_v, idx_t_v, ce_rows, te_rows, out_v, sem):
    wid = lax.axis_index("s") * NC + lax.axis_index("c")
    iota = lax.iota(jnp.int32, 16)

    for c in range(NCHUNK):
        ck = wid * NCHUNK + c
        pltpu.sync_copy(ctx_idx.at[pl.ds(ck * CB, CB)], idx_c_v)
        for j in range(NT):
            pltpu.sync_copy(tgt_idx.at[pl.ds((ck * NT + j) * CB, CB)],
                            idx_t_v.at[j])
        copies = [pltpu.async_copy(ctx_table.at[idx_c_v], ce_rows, sem)]
        for j in range(NT):
            copies.append(pltpu.async_copy(
                tgt_table.at[idx_t_v.at[j]],
                te_rows.at[pl.ds(j * CB, CB)], sem))
        for cp in copies:
            cp.wait()

        for g in range(GRP):
            b0 = g * 16
            row_c = iota + b0
            row_t = (iota + b0) * NT

            def body(d, accs, row_c=row_c, row_t=row_t):
                dsp = jnp.full((16,), d, jnp.int32)
                ce_col = plsc.load_gather(ce_rows, [row_c, dsp])
                return tuple(
                    accs[n] + plsc.load_gather(te_rows, [row_t + n, dsp]) * ce_col
                    for n in range(NT))

            zero = jnp.zeros((16,), jnp.float32)
            accs = lax.fori_loop(0, DIM, body, (zero,) * NT)
            for n in range(NT):
                plsc.store_scatter(out_v, [row_t + n], accs[n])

        pltpu.sync_copy(out_v, out.at[pl.ds(ck * CB * NT, CB * NT)])


def kernel(context, target, ctx_table, tgt_table):
    ctx_idx = context.reshape(BATCH)
    tgt_idx = target.reshape(BATCH * NT)

    mesh = plsc.VectorSubcoreMesh(core_axis_name="c", subcore_axis_name="s",
                                  num_cores=NC, num_subcores=NS)
    run = pl.kernel(
        _cbow_body,
        out_type=jax.ShapeDtypeStruct((BATCH * NT,), jnp.float32),
        mesh=mesh,
        compiler_params=pltpu.CompilerParams(needs_layout_passes=False,
                                             use_tc_tiling_on_sc=False),
        scratch_types=[
            pltpu.VMEM((CB,), jnp.int32),
            pltpu.VMEM((NT, CB), jnp.int32),
            pltpu.VMEM((CB, DIM), jnp.float32),
            pltpu.VMEM((NT * CB, DIM), jnp.float32),
            pltpu.VMEM((CB * NT,), jnp.float32),
            pltpu.SemaphoreType.DMA,
        ],
    )
    out_flat = run(ctx_idx, tgt_idx, ctx_table, tgt_table)
    return out_flat.reshape(BATCH, NT)


if __name__ == "__main__":
    pass
